# baseline (speedup 1.0000x reference)
import functools

import jax
import jax.numpy as jnp
from jax.experimental import pallas as pl
from jax.experimental.pallas import tpu as pltpu

_TB = 8192


def _fused_kernel(x_ref, w_ref, b_ref, y_ref, reg_ref, *, l1, l2):
    w = w_ref[...]
    y_ref[...] = jnp.sum(x_ref[...] * w, axis=1, keepdims=True) + b_ref[0]
    reg_ref[...] = (l1 * jnp.sum(jnp.abs(w)) + l2 * jnp.sqrt(jnp.sum(w * w))).reshape(
        1, 1
    )


def kernel(x, weight, bias):
    B, D = x.shape
    tb = min(_TB, B)
    grid = (pl.cdiv(B, tb),)

    y, reg = pl.pallas_call(
        functools.partial(_fused_kernel, l1=0.01, l2=0.01),
        grid=grid,
        in_specs=[
            pl.BlockSpec((tb, D), lambda i: (i, 0)),
            pl.BlockSpec((1, D), lambda i: (0, 0)),
            pl.BlockSpec(memory_space=pltpu.MemorySpace.SMEM),
        ],
        out_specs=[
            pl.BlockSpec((tb, 1), lambda i: (i, 0)),
            pl.BlockSpec((1, 1), lambda i: (0, 0)),
        ],
        out_shape=[
            jax.ShapeDtypeStruct((B, 1), jnp.float32),
            jax.ShapeDtypeStruct((1, 1), jnp.float32),
        ],
        compiler_params=pltpu.CompilerParams(
            dimension_semantics=("parallel",),
            vmem_limit_bytes=64 * 1024 * 1024,
        ),
    )(x, weight, bias)
    return y, reg[0, 0]


# speedup vs baseline: 1.0292x; 1.0292x over previous
import functools

import jax
import jax.numpy as jnp
from jax.experimental import pallas as pl
from jax.experimental.pallas import tpu as pltpu

_TB = 4096


def _fused_kernel(x_ref, w_ref, b_ref, y_ref, reg_ref, *, l1, l2):
    w = w_ref[...]
    y_ref[...] = jnp.sum(x_ref[...] * w, axis=1, keepdims=True) + b_ref[0]
    reg_ref[...] = (l1 * jnp.sum(jnp.abs(w)) + l2 * jnp.sqrt(jnp.sum(w * w))).reshape(
        1, 1
    )


def kernel(x, weight, bias):
    B, D = x.shape
    tb = min(_TB, B)
    grid = (pl.cdiv(B, tb),)

    y, reg = pl.pallas_call(
        functools.partial(_fused_kernel, l1=0.01, l2=0.01),
        grid=grid,
        in_specs=[
            pl.BlockSpec((tb, D), lambda i: (i, 0)),
            pl.BlockSpec((1, D), lambda i: (0, 0)),
            pl.BlockSpec(memory_space=pltpu.MemorySpace.SMEM),
        ],
        out_specs=[
            pl.BlockSpec((tb, 1), lambda i: (i, 0)),
            pl.BlockSpec((1, 1), lambda i: (0, 0)),
        ],
        out_shape=[
            jax.ShapeDtypeStruct((B, 1), jnp.float32),
            jax.ShapeDtypeStruct((1, 1), jnp.float32),
        ],
        compiler_params=pltpu.CompilerParams(
            dimension_semantics=("parallel",),
            vmem_limit_bytes=64 * 1024 * 1024,
        ),
    )(x, weight, bias)
    return y, reg[0, 0]
